# baseline (speedup 1.0000x reference)
import functools

import jax
import jax.numpy as jnp
from jax import lax
from jax.experimental import pallas as pl
from jax.experimental.pallas import tpu as pltpu
from jax.experimental.pallas import tpu_sc as plsc

IOU_T = 0.4
CONF_T = 0.2
C = 128
B, N = 4, 5000
NC = (N + C - 1) // C
NPAD = NC * C
NW = 32
UNITS = B * NC
UPW = UNITS // NW


def _sc_worker_id():
    return lax.axis_index("s") * 2 + lax.axis_index("c")


def _gather_pack_kernel(boxes_hbm, scores_hbm, ordc_hbm, pk_hbm,
                        idx_v, idx2_v, val_v, pk_v, sem):
    wid = _sc_worker_id()

    for t in range(UPW):
        u = wid * UPW + t
        k = lax.rem(u, NC)
        pltpu.sync_copy(ordc_hbm.at[pl.ds(u * C, C)], idx_v)

        pltpu.async_copy(scores_hbm.at[idx_v], val_v, sem).wait()
        for i in range(C // 16):
            pos = lax.iota(jnp.int32, 16) + (k * C + 16 * i)
            real = pos < N
            sv = val_v[pl.ds(16 * i, 16)]
            pk_v[pl.ds(4 * C + 16 * i, 16)] = jnp.where(real, sv, -1.0)

        for c in range(4):
            for i in range(C // 16):
                ov = idx_v[pl.ds(16 * i, 16)]
                idx2_v[pl.ds(16 * i, 16)] = ov * 4 + c
            pltpu.async_copy(boxes_hbm.at[idx2_v], val_v, sem).wait()
            for i in range(C // 16):
                pos = lax.iota(jnp.int32, 16) + (k * C + 16 * i)
                real = pos < N
                cv = jnp.where(real, val_v[pl.ds(16 * i, 16)], 0.0)
                pk_v[pl.ds(c * C + 16 * i, 16)] = cv

        for i in range(C // 16):
            w = (pk_v[pl.ds(2 * C + 16 * i, 16)]
                 - pk_v[pl.ds(0 * C + 16 * i, 16)])
            h = (pk_v[pl.ds(3 * C + 16 * i, 16)]
                 - pk_v[pl.ds(1 * C + 16 * i, 16)])
            pk_v[pl.ds(5 * C + 16 * i, 16)] = w * h

        pltpu.sync_copy(pk_v.at[pl.ds(0, 6 * C)],
                        pk_hbm.at[pl.ds(u * 8 * C, 6 * C)])


def _scatter_out_kernel(pk_hbm, keep_hbm, ords_hbm, out_hbm,
                        idx_v, idx2_v, pk_v, keep_v, val_v, sem):
    wid = _sc_worker_id()

    for t in range(UPW):
        u = wid * UPW + t
        pltpu.sync_copy(ords_hbm.at[pl.ds(u * C, C)], idx_v)
        pltpu.sync_copy(keep_hbm.at[pl.ds(u * C, C)], keep_v)
        pltpu.sync_copy(pk_hbm.at[pl.ds(u * 8 * C, 6 * C)],
                        pk_v.at[pl.ds(0, 6 * C)])
        for c in range(5):
            for i in range(C // 16):
                tv = idx_v[pl.ds(16 * i, 16)]
                idx2_v[pl.ds(16 * i, 16)] = tv * 5 + c
                kf = keep_v[pl.ds(16 * i, 16)]
                val_v[pl.ds(16 * i, 16)] = kf * pk_v[pl.ds(c * C + 16 * i, 16)]
            pltpu.async_copy(val_v, out_hbm.at[idx2_v], sem).wait()


_sc_mesh = plsc.VectorSubcoreMesh(core_axis_name="c", subcore_axis_name="s")

_gather_pack = functools.partial(
    pl.kernel, mesh=_sc_mesh,
    out_type=jax.ShapeDtypeStruct((UNITS * 8 * C,), jnp.float32),
    scratch_types=[
        pltpu.VMEM((C,), jnp.int32),
        pltpu.VMEM((C,), jnp.int32),
        pltpu.VMEM((C,), jnp.float32),
        pltpu.VMEM((8 * C,), jnp.float32),
        pltpu.SemaphoreType.DMA,
    ],
)(_gather_pack_kernel)

_scatter_out = functools.partial(
    pl.kernel, mesh=_sc_mesh,
    out_type=jax.ShapeDtypeStruct((B * N * 5 + 64,), jnp.float32),
    scratch_types=[
        pltpu.VMEM((C,), jnp.int32),
        pltpu.VMEM((C,), jnp.int32),
        pltpu.VMEM((8 * C,), jnp.float32),
        pltpu.VMEM((C,), jnp.float32),
        pltpu.VMEM((C,), jnp.float32),
        pltpu.SemaphoreType.DMA,
    ],
)(_scatter_out_kernel)


def _nms_body(pk_ref, keep_ref, sup_ref):
    nc = pk_ref.shape[1]
    rows = jax.lax.broadcasted_iota(jnp.int32, (C, C), 0)
    cols = jax.lax.broadcasted_iota(jnp.int32, (C, C), 1)

    sup_ref[...] = jnp.zeros_like(sup_ref)

    def iou_gt(x1r, y1r, x2r, y2r, ar, x1c, y1c, x2c, y2c, ac):
        xx1 = jnp.maximum(x1r, x1c)
        yy1 = jnp.maximum(y1r, y1c)
        xx2 = jnp.minimum(x2r, x2c)
        yy2 = jnp.minimum(y2r, y2c)
        inter = jnp.clip(xx2 - xx1, 0.0) * jnp.clip(yy2 - yy1, 0.0)
        iou = inter / (ar + ac - inter + 1e-9)
        return iou > IOU_T

    def chunk_body(k, _):
        blk = pk_ref[0, k]
        blkt = jnp.transpose(blk)
        x1 = blk[0:1]
        y1 = blk[1:2]
        x2 = blk[2:3]
        y2 = blk[3:4]
        s = blk[4:5]
        ar = blk[5:6]
        x1t = blkt[:, 0:1]
        y1t = blkt[:, 1:2]
        x2t = blkt[:, 2:3]
        y2t = blkt[:, 3:4]
        st = blkt[:, 4:5]
        art = blkt[:, 5:6]

        g = iou_gt(x1t, y1t, x2t, y2t, art, x1, y1, x2, y2, ar)
        gt = jnp.transpose(g)
        s_rl = (g & (rows < cols)).astype(jnp.float32)
        s_lr = (gt & (cols < rows)).astype(jnp.float32)

        pre = sup_ref[k] > 0
        valid_l = (s > CONF_T) & ~pre
        valid_r = (st > CONF_T) & ~jnp.transpose(pre)
        valid_lf = valid_l.astype(jnp.float32)
        valid_rf = valid_r.astype(jnp.float32)

        def fix_cond(st_):
            return st_[0]

        def fix_body(st_):
            _, keep_lf = st_
            sup_r = jnp.max(s_lr * keep_lf, axis=1, keepdims=True)
            keep_rf_ = valid_rf * (1.0 - jnp.minimum(sup_r, 1.0))
            sup_l = jnp.max(s_rl * keep_rf_, axis=0, keepdims=True)
            keep_lf2 = valid_lf * (1.0 - jnp.minimum(sup_l, 1.0))
            changed = jnp.any(keep_lf2 != keep_lf)
            return changed, keep_lf2

        _, keep_lf = jax.lax.while_loop(
            fix_cond, fix_body, (jnp.bool_(True), valid_lf))
        keep_l = keep_lf > 0.0
        keep_rf = valid_rf * (1.0 - jnp.minimum(
            jnp.max(s_lr * keep_lf, axis=1, keepdims=True), 1.0))

        tiny = ((x2 - x1) >= 1.0) & ((y2 - y1) >= 1.0)
        keep_ref[0, k] = (keep_l & tiny).astype(jnp.float32)

        def cross_body(m, _):
            b2 = pk_ref[0, m]
            cs = iou_gt(x1t, y1t, x2t, y2t, art,
                        b2[0:1], b2[1:2], b2[2:3], b2[3:4], b2[5:6])
            supm = jnp.max(cs.astype(jnp.float32) * keep_rf,
                           axis=0, keepdims=True)
            sup_ref[m] = jnp.maximum(sup_ref[m], supm)
            return 0

        jax.lax.fori_loop(k + 1, nc, cross_body, 0)
        return 0

    jax.lax.fori_loop(0, nc, chunk_body, 0)


@jax.jit
def kernel(boxes, scores):
    order = jnp.argsort(-scores, axis=1)

    pad = NPAD - N
    bofs = (jnp.arange(B, dtype=jnp.int32) * N)[:, None]
    ordc = jnp.pad(order.astype(jnp.int32), ((0, 0), (0, pad))) + bofs
    spare = jnp.full((B, NPAD), B * N, jnp.int32)
    mask_pad = jnp.arange(NPAD, dtype=jnp.int32)[None, :] >= N
    ords = jnp.where(mask_pad, spare, ordc).reshape(-1)
    ordc = jnp.minimum(ordc, (B * N) - 1).reshape(-1)

    pk_flat = _gather_pack(boxes.reshape(-1), scores.reshape(-1), ordc)
    pk = pk_flat.reshape(B, NC, 8, C)

    keep_sorted = pl.pallas_call(
        _nms_body,
        grid=(B,),
        in_specs=[pl.BlockSpec((1, NC, 8, C), lambda b: (b, 0, 0, 0))],
        out_specs=pl.BlockSpec((1, NC, 1, C), lambda b: (b, 0, 0, 0)),
        out_shape=jax.ShapeDtypeStruct((B, NC, 1, C), jnp.float32),
        scratch_shapes=[pltpu.VMEM((NC, 1, C), jnp.float32)],
    )(pk)

    out_flat = _scatter_out(pk_flat, keep_sorted.reshape(-1), ords)
    return out_flat[:B * N * 5].reshape(B, N, 5)


# speedup vs baseline: 2.0558x; 2.0558x over previous
import functools

import jax
import jax.numpy as jnp
from jax import lax
from jax.experimental import pallas as pl
from jax.experimental.pallas import tpu as pltpu
from jax.experimental.pallas import tpu_sc as plsc

IOU_T = 0.4
CONF_T = 0.2
C = 128
B, N = 4, 5000
NC = (N + C - 1) // C
NPAD = NC * C
NTAIL = N - (NC - 1) * C
NW = 32
UNITS = B * NC
UPW = UNITS // NW


def _sc_worker_id():
    return lax.axis_index("s") * 2 + lax.axis_index("c")


def _gather_pack_kernel(boxes_hbm, scores_hbm, ordc_hbm, pk_hbm,
                        idx_v, idx2_v, val_v, pk_v, sem):
    wid = _sc_worker_id()

    for t in range(UPW):
        u = wid * UPW + t
        k = lax.rem(u, NC)
        pltpu.sync_copy(ordc_hbm.at[pl.ds(u * C, C)], idx_v)

        pltpu.async_copy(scores_hbm.at[idx_v], val_v, sem).wait()
        for i in range(C // 16):
            pos = lax.iota(jnp.int32, 16) + (k * C + 16 * i)
            real = pos < N
            sv = val_v[pl.ds(16 * i, 16)]
            pk_v[pl.ds(4 * C + 16 * i, 16)] = jnp.where(real, sv, -1.0)

        for c in range(4):
            for i in range(C // 16):
                ov = idx_v[pl.ds(16 * i, 16)]
                idx2_v[pl.ds(16 * i, 16)] = ov * 4 + c
            pltpu.async_copy(boxes_hbm.at[idx2_v], val_v, sem).wait()
            for i in range(C // 16):
                pos = lax.iota(jnp.int32, 16) + (k * C + 16 * i)
                real = pos < N
                cv = jnp.where(real, val_v[pl.ds(16 * i, 16)], 0.0)
                pk_v[pl.ds(c * C + 16 * i, 16)] = cv

        for i in range(C // 16):
            w = (pk_v[pl.ds(2 * C + 16 * i, 16)]
                 - pk_v[pl.ds(0 * C + 16 * i, 16)])
            h = (pk_v[pl.ds(3 * C + 16 * i, 16)]
                 - pk_v[pl.ds(1 * C + 16 * i, 16)])
            pk_v[pl.ds(5 * C + 16 * i, 16)] = w * h

        pltpu.sync_copy(pk_v.at[pl.ds(0, 6 * C)],
                        pk_hbm.at[pl.ds(u * 8 * C, 6 * C)])


def _keep_scatter_kernel(keep_hbm, ords_hbm, ko_hbm, idx_v, keep_v, sem):
    wid = _sc_worker_id()
    for t in range(UPW):
        u = wid * UPW + t
        pltpu.sync_copy(ords_hbm.at[pl.ds(u * C, C)], idx_v)
        pltpu.sync_copy(keep_hbm.at[pl.ds(u * C, C)], keep_v)
        pltpu.async_copy(keep_v, ko_hbm.at[idx_v], sem).wait()


_sc_mesh = plsc.VectorSubcoreMesh(core_axis_name="c", subcore_axis_name="s")

_gather_pack = functools.partial(
    pl.kernel, mesh=_sc_mesh,
    out_type=jax.ShapeDtypeStruct((UNITS * 8 * C,), jnp.float32),
    scratch_types=[
        pltpu.VMEM((C,), jnp.int32),
        pltpu.VMEM((C,), jnp.int32),
        pltpu.VMEM((C,), jnp.float32),
        pltpu.VMEM((8 * C,), jnp.float32),
        pltpu.SemaphoreType.DMA,
    ],
)(_gather_pack_kernel)

_keep_scatter = functools.partial(
    pl.kernel, mesh=_sc_mesh,
    out_type=jax.ShapeDtypeStruct((B * N + 64,), jnp.float32),
    scratch_types=[
        pltpu.VMEM((C,), jnp.int32),
        pltpu.VMEM((C,), jnp.float32),
        pltpu.SemaphoreType.DMA,
    ],
)(_keep_scatter_kernel)


def _nms_body(pk_ref, keep_ref, sup_ref):
    nc = pk_ref.shape[1]
    rows = jax.lax.broadcasted_iota(jnp.int32, (C, C), 0)
    cols = jax.lax.broadcasted_iota(jnp.int32, (C, C), 1)

    sup_ref[...] = jnp.zeros_like(sup_ref)

    smax = jnp.max(pk_ref[0, :, 4:5, :], axis=2)
    kc = jnp.sum((smax > CONF_T).astype(jnp.int32))

    def iou_gt(x1r, y1r, x2r, y2r, ar, x1c, y1c, x2c, y2c, ac):
        xx1 = jnp.maximum(x1r, x1c)
        yy1 = jnp.maximum(y1r, y1c)
        xx2 = jnp.minimum(x2r, x2c)
        yy2 = jnp.minimum(y2r, y2c)
        inter = jnp.clip(xx2 - xx1, 0.0) * jnp.clip(yy2 - yy1, 0.0)
        iou = inter / (ar + ac - inter + 1e-9)
        return iou > IOU_T

    def chunk_body(k, _):
        blk = pk_ref[0, k]
        blkt = jnp.transpose(blk)
        x1 = blk[0:1]
        y1 = blk[1:2]
        x2 = blk[2:3]
        y2 = blk[3:4]
        s = blk[4:5]
        ar = blk[5:6]
        x1t = blkt[:, 0:1]
        y1t = blkt[:, 1:2]
        x2t = blkt[:, 2:3]
        y2t = blkt[:, 3:4]
        st = blkt[:, 4:5]
        art = blkt[:, 5:6]

        g = iou_gt(x1t, y1t, x2t, y2t, art, x1, y1, x2, y2, ar)
        gt = jnp.transpose(g)
        s_rl = (g & (rows < cols)).astype(jnp.float32)
        s_lr = (gt & (cols < rows)).astype(jnp.float32)

        pre = sup_ref[k] > 0
        valid_l = (s > CONF_T) & ~pre
        valid_r = (st > CONF_T) & ~jnp.transpose(pre)
        valid_lf = valid_l.astype(jnp.float32)
        valid_rf = valid_r.astype(jnp.float32)

        def fix_cond(st_):
            return st_[0]

        def fix_body(st_):
            _, keep_lf = st_
            sup_r = jnp.max(s_lr * keep_lf, axis=1, keepdims=True)
            keep_rf_ = valid_rf * (1.0 - sup_r)
            sup_l = jnp.max(s_rl * keep_rf_, axis=0, keepdims=True)
            keep_lf2 = valid_lf * (1.0 - sup_l)
            changed = jnp.any(keep_lf2 != keep_lf)
            return changed, keep_lf2

        _, keep_lf = jax.lax.while_loop(
            fix_cond, fix_body, (jnp.bool_(True), valid_lf))
        keep_l = keep_lf > 0.0
        keep_rf = valid_rf * (1.0 - jnp.max(s_lr * keep_lf,
                                            axis=1, keepdims=True))

        tiny = ((x2 - x1) >= 1.0) & ((y2 - y1) >= 1.0)
        keep_ref[0, k] = (keep_l & tiny).astype(jnp.float32)

        def cross_tile(m):
            b2 = pk_ref[0, m]
            cs = iou_gt(x1t, y1t, x2t, y2t, art,
                        b2[0:1], b2[1:2], b2[2:3], b2[3:4], b2[5:6])
            return jnp.max(cs.astype(jnp.float32) * keep_rf,
                           axis=0, keepdims=True)

        def cross_body(ii, _):
            m = k + 1 + 2 * ii
            sup_ref[m] = jnp.maximum(sup_ref[m], cross_tile(m))
            m2r = jnp.minimum(m + 1, nc - 1)
            m2w = jnp.where(m + 1 < kc, m + 1, nc)
            sup_ref[m2w] = jnp.maximum(sup_ref[m2w], cross_tile(m2r))
            return 0

        npairs = jnp.maximum(kc - (k + 1), 0)
        jax.lax.fori_loop(0, (npairs + 1) // 2, cross_body, 0)
        return 0

    jax.lax.fori_loop(0, nc, chunk_body, 0)


def _mask_out_body(bx_ref, sc_ref, kf_ref, out_ref):
    kf = kf_ref[0]
    bx = bx_ref[0] * kf
    sv = sc_ref[0] * kf
    out_ref[0] = jnp.concatenate([bx, sv], axis=1)


@jax.jit
def kernel(boxes, scores):
    order = jnp.argsort(-scores, axis=1)

    pad = NPAD - N
    bofs = (jnp.arange(B, dtype=jnp.int32) * N)[:, None]
    ordc = jnp.pad(order.astype(jnp.int32), ((0, 0), (0, pad))) + bofs
    spare = jnp.full((B, NPAD), B * N, jnp.int32)
    mask_pad = jnp.arange(NPAD, dtype=jnp.int32)[None, :] >= N
    ords = jnp.where(mask_pad, spare, ordc).reshape(-1)
    ordc = jnp.minimum(ordc, (B * N) - 1).reshape(-1)

    boxes_flat = boxes.reshape(-1)
    scores_flat = scores.reshape(-1)
    pk_flat = _gather_pack(boxes_flat, scores_flat, ordc)
    pk = pk_flat.reshape(B, NC, 8, C)

    keep_sorted = pl.pallas_call(
        _nms_body,
        grid=(B,),
        in_specs=[pl.BlockSpec((1, NC, 8, C), lambda b: (b, 0, 0, 0))],
        out_specs=pl.BlockSpec((1, NC, 1, C), lambda b: (b, 0, 0, 0)),
        out_shape=jax.ShapeDtypeStruct((B, NC, 1, C), jnp.float32),
        scratch_shapes=[pltpu.VMEM((NC + 1, 1, C), jnp.float32)],
    )(pk)

    keep_orig = _keep_scatter(keep_sorted.reshape(-1), ords)
    kf = keep_orig[:B * N].reshape(B, N, 1)
    out = pl.pallas_call(
        _mask_out_body,
        grid=(B,),
        in_specs=[pl.BlockSpec((1, N, 4), lambda b: (b, 0, 0)),
                  pl.BlockSpec((1, N, 1), lambda b: (b, 0, 0)),
                  pl.BlockSpec((1, N, 1), lambda b: (b, 0, 0))],
        out_specs=pl.BlockSpec((1, N, 5), lambda b: (b, 0, 0)),
        out_shape=jax.ShapeDtypeStruct((B, N, 5), jnp.float32),
    )(boxes, scores[..., None], kf)
    return out
